# initial kernel scaffold (unmeasured)
import jax
import jax.numpy as jnp
from jax import lax
from jax.experimental import pallas as pl
from jax.experimental.pallas import tpu as pltpu

N_DEV = 8
N_TOK = 1024
D_IN = 256
D_OUT = 512
E_LOCAL = 4
ROWS = N_TOK // N_DEV


def kernel(x, router_W, route_idx, expert_W, shared_W):
    def body(x_ref, rw_ref, idx_ref, ew_ref, sw_ref, out_ref,
             stage_ref, comm_ref, send_sems, recv_sems):
        my = lax.axis_index("i")

        comm_ref[...] = jnp.zeros((N_DEV, ROWS, D_OUT), jnp.bfloat16)

        barrier_sem = pltpu.get_barrier_semaphore()
        for off in range(1, N_DEV):
            pl.semaphore_signal(
                barrier_sem, inc=1,
                device_id=((my + off) % N_DEV,),
                device_id_type=pl.DeviceIdType.MESH,
            )
        pl.semaphore_wait(barrier_sem, N_DEV - 1)

        x = x_ref[...]
        scores = jnp.dot(x.astype(jnp.bfloat16), rw_ref[...].astype(jnp.bfloat16),
                         preferred_element_type=jnp.float32)
        m = jnp.max(scores, axis=-1, keepdims=True)
        p = jnp.exp(scores - m)
        p = p / jnp.sum(p, axis=-1, keepdims=True)
        gates = lax.dynamic_slice(p, (0, my * E_LOCAL), (N_TOK, E_LOCAL))
        idx = idx_ref[...]
        local_e = my * E_LOCAL + lax.broadcasted_iota(jnp.int32, (1, E_LOCAL), 1)
        coeff = jnp.where(idx == local_e, gates, 0.0)

        xbig = jnp.concatenate(
            [(x * coeff[:, k:k + 1]).astype(jnp.bfloat16) for k in range(E_LOCAL)],
            axis=1)
        wbig = ew_ref[...].reshape(E_LOCAL * D_IN, D_OUT).astype(jnp.bfloat16)
        partial = jnp.dot(xbig, wbig, preferred_element_type=jnp.float32)

        stage_ref[...] = partial.astype(jnp.bfloat16).reshape(N_DEV, ROWS, D_OUT)

        rdmas = []
        for off in range(1, N_DEV):
            o = (my + off) % N_DEV
            rdma = pltpu.make_async_remote_copy(
                src_ref=stage_ref.at[o],
                dst_ref=comm_ref.at[my],
                send_sem=send_sems.at[off],
                recv_sem=recv_sems.at[my],
                device_id=(o,),
                device_id_type=pl.DeviceIdType.MESH,
            )
            rdma.start()
            rdmas.append(rdma)

        x_my = lax.dynamic_slice(x, (my * ROWS, 0), (ROWS, D_IN))
        shared_local = jnp.dot(x_my.astype(jnp.bfloat16),
                               sw_ref[...].astype(jnp.bfloat16),
                               preferred_element_type=jnp.float32)
        own = lax.dynamic_slice(partial, (my * ROWS, 0), (ROWS, D_OUT))

        for off in range(1, N_DEV):
            s = (my + off) % N_DEV
            recv = pltpu.make_async_remote_copy(
                src_ref=stage_ref.at[s],
                dst_ref=comm_ref.at[s],
                send_sem=send_sems.at[off],
                recv_sem=recv_sems.at[s],
                device_id=(s,),
                device_id_type=pl.DeviceIdType.MESH,
            )
            recv.wait_recv()

        total = jnp.sum(comm_ref[...].astype(jnp.float32), axis=0)
        out_ref[...] = total + own + shared_local

        for rdma in rdmas:
            rdma.wait_send()

    return pl.pallas_call(
        body,
        out_shape=jax.ShapeDtypeStruct((ROWS, D_OUT), jnp.float32),
        in_specs=[pl.BlockSpec(memory_space=pltpu.VMEM)] * 5,
        out_specs=pl.BlockSpec(memory_space=pltpu.VMEM),
        scratch_shapes=[
            pltpu.VMEM((N_DEV, ROWS, D_OUT), jnp.bfloat16),
            pltpu.VMEM((N_DEV, ROWS, D_OUT), jnp.bfloat16),
            pltpu.SemaphoreType.DMA((N_DEV,)),
            pltpu.SemaphoreType.DMA((N_DEV,)),
        ],
        compiler_params=pltpu.CompilerParams(collective_id=0),
    )(x, router_W, route_idx, expert_W, shared_W)


# baseline (device time: 21494 ns/iter reference)
import jax
import jax.numpy as jnp
from jax import lax
from jax.experimental import pallas as pl
from jax.experimental.pallas import tpu as pltpu

N_DEV = 8
N_TOK = 1024
D_IN = 256
D_OUT = 512
E_LOCAL = 4
ROWS = N_TOK // N_DEV


def kernel(x, router_W, route_idx, expert_W, shared_W):
    def body(x_ref, rw_ref, idx_ref, ew_ref, sw_ref, out_ref,
             stage_ref, comm_ref, send_sems, recv_sems):
        my = lax.axis_index("i")

        comm_ref[...] = jnp.zeros((N_DEV, ROWS, D_OUT), jnp.bfloat16)

        barrier_sem = pltpu.get_barrier_semaphore()
        for off in range(1, N_DEV):
            pl.semaphore_signal(
                barrier_sem, inc=1,
                device_id=((my + off) % N_DEV,),
                device_id_type=pl.DeviceIdType.MESH,
            )
        pl.semaphore_wait(barrier_sem, N_DEV - 1)

        x = x_ref[...]
        scores = jnp.dot(x.astype(jnp.bfloat16), rw_ref[...].astype(jnp.bfloat16),
                         preferred_element_type=jnp.float32)
        m = jnp.max(scores, axis=-1, keepdims=True)
        p = jnp.exp(scores - m)
        p = p / jnp.sum(p, axis=-1, keepdims=True)
        idx = idx_ref[...]
        all_e = lax.broadcasted_iota(jnp.int32, (1, 32), 1)
        g = jnp.sum(jnp.where(idx == all_e, p, 0.0), axis=1, keepdims=True)
        local_e = my * E_LOCAL + lax.broadcasted_iota(jnp.int32, (1, E_LOCAL), 1)
        coeff = jnp.where(idx == local_e, g, 0.0)

        xbig = jnp.concatenate(
            [(x * coeff[:, k:k + 1]).astype(jnp.bfloat16) for k in range(E_LOCAL)],
            axis=1)
        wbig = ew_ref[...].reshape(E_LOCAL * D_IN, D_OUT).astype(jnp.bfloat16)
        partial = jnp.dot(xbig, wbig, preferred_element_type=jnp.float32)

        stage_ref[...] = partial.astype(jnp.bfloat16).reshape(N_DEV, ROWS, D_OUT)

        rdmas = []
        for off in range(1, N_DEV):
            o = (my + off) % N_DEV
            rdma = pltpu.make_async_remote_copy(
                src_ref=stage_ref.at[o],
                dst_ref=comm_ref.at[my],
                send_sem=send_sems.at[off],
                recv_sem=recv_sems.at[my],
                device_id=(o,),
                device_id_type=pl.DeviceIdType.MESH,
            )
            rdma.start()
            rdmas.append(rdma)

        x_my = x_ref[pl.ds(my * ROWS, ROWS), :]
        shared_local = jnp.dot(x_my.astype(jnp.bfloat16),
                               sw_ref[...].astype(jnp.bfloat16),
                               preferred_element_type=jnp.float32)
        own = stage_ref[pl.ds(my, 1)][0].astype(jnp.float32)

        for off in range(1, N_DEV):
            s = (my + off) % N_DEV
            recv = pltpu.make_async_remote_copy(
                src_ref=stage_ref.at[s],
                dst_ref=comm_ref.at[s],
                send_sem=send_sems.at[off],
                recv_sem=recv_sems.at[s],
                device_id=(s,),
                device_id_type=pl.DeviceIdType.MESH,
            )
            recv.wait_recv()

        total = jnp.sum(comm_ref[...].astype(jnp.float32), axis=0)
        out_ref[...] = total + own + shared_local

        for rdma in rdmas:
            rdma.wait_send()

    return pl.pallas_call(
        body,
        out_shape=jax.ShapeDtypeStruct((ROWS, D_OUT), jnp.float32),
        in_specs=[pl.BlockSpec(memory_space=pltpu.VMEM)] * 5,
        out_specs=pl.BlockSpec(memory_space=pltpu.VMEM),
        scratch_shapes=[
            pltpu.VMEM((N_DEV, ROWS, D_OUT), jnp.bfloat16),
            pltpu.VMEM((N_DEV, ROWS, D_OUT), jnp.bfloat16),
            pltpu.SemaphoreType.DMA((N_DEV,)),
            pltpu.SemaphoreType.DMA((N_DEV,)),
        ],
        compiler_params=pltpu.CompilerParams(collective_id=0),
    )(x, router_W, route_idx, expert_W, shared_W)


# device time: 20315 ns/iter; 1.0580x vs baseline; 1.0580x over previous
import jax
import jax.numpy as jnp
from jax import lax
from jax.experimental import pallas as pl
from jax.experimental.pallas import tpu as pltpu

N_DEV = 8
N_TOK = 1024
D_IN = 256
D_OUT = 512
E_LOCAL = 4
ROWS = N_TOK // N_DEV


def kernel(x, router_W, route_idx, expert_W, shared_W):
    def body(x_ref, rw_ref, idx_ref, ew_ref, sw_ref, out_ref,
             stage_ref, comm_ref, coeff_ref, send_sems, recv_sems):
        my = lax.axis_index("i")

        barrier_sem = pltpu.get_barrier_semaphore()
        for off in range(1, N_DEV):
            pl.semaphore_signal(
                barrier_sem, inc=1,
                device_id=((my + off) % N_DEV,),
                device_id_type=pl.DeviceIdType.MESH,
            )
        pl.semaphore_wait(barrier_sem, N_DEV - 1)

        x = x_ref[...]
        scores = jnp.dot(x.astype(jnp.bfloat16), rw_ref[...].astype(jnp.bfloat16),
                         preferred_element_type=jnp.float32)
        m = jnp.max(scores, axis=-1, keepdims=True)
        p = jnp.exp(scores - m)
        p = p / jnp.sum(p, axis=-1, keepdims=True)
        idx = idx_ref[...]
        all_e = lax.broadcasted_iota(jnp.int32, (1, 32), 1)
        g = jnp.sum(jnp.where(idx == all_e, p, 0.0), axis=1, keepdims=True)
        local_e = my * E_LOCAL + lax.broadcasted_iota(jnp.int32, (1, E_LOCAL), 1)
        coeff_ref[...] = jnp.where(idx == local_e, g, 0.0)

        wbig = ew_ref[...].reshape(E_LOCAL * D_IN, D_OUT).astype(jnp.bfloat16)

        rdmas = []
        for off in range(1, N_DEV):
            o = (my + off) % N_DEV
            xc = x_ref[pl.ds(o * ROWS, ROWS), :]
            cc = coeff_ref[pl.ds(o * ROWS, ROWS), :]
            xbig = jnp.concatenate(
                [(xc * cc[:, k:k + 1]).astype(jnp.bfloat16)
                 for k in range(E_LOCAL)], axis=1)
            chunk = jnp.dot(xbig, wbig, preferred_element_type=jnp.float32)
            stage_ref[pl.ds(o, 1)] = chunk.astype(jnp.bfloat16)[None]
            rdma = pltpu.make_async_remote_copy(
                src_ref=stage_ref.at[o],
                dst_ref=comm_ref.at[my],
                send_sem=send_sems.at[off],
                recv_sem=recv_sems.at[my],
                device_id=(o,),
                device_id_type=pl.DeviceIdType.MESH,
            )
            rdma.start()
            rdmas.append(rdma)

        xm = x_ref[pl.ds(my * ROWS, ROWS), :]
        cm = coeff_ref[pl.ds(my * ROWS, ROWS), :]
        xbig_m = jnp.concatenate(
            [(xm * cm[:, k:k + 1]).astype(jnp.bfloat16)
             for k in range(E_LOCAL)], axis=1)
        own = jnp.dot(xbig_m, wbig, preferred_element_type=jnp.float32)
        shared_local = jnp.dot(xm.astype(jnp.bfloat16),
                               sw_ref[...].astype(jnp.bfloat16),
                               preferred_element_type=jnp.float32)
        acc = own + shared_local

        for off in range(1, N_DEV):
            s = (my + off) % N_DEV
            recv = pltpu.make_async_remote_copy(
                src_ref=stage_ref.at[s],
                dst_ref=comm_ref.at[s],
                send_sem=send_sems.at[off],
                recv_sem=recv_sems.at[s],
                device_id=(s,),
                device_id_type=pl.DeviceIdType.MESH,
            )
            recv.wait_recv()
            acc = acc + comm_ref[pl.ds(s, 1)][0].astype(jnp.float32)

        out_ref[...] = acc

        for rdma in rdmas:
            rdma.wait_send()

    return pl.pallas_call(
        body,
        out_shape=jax.ShapeDtypeStruct((ROWS, D_OUT), jnp.float32),
        in_specs=[pl.BlockSpec(memory_space=pltpu.VMEM)] * 5,
        out_specs=pl.BlockSpec(memory_space=pltpu.VMEM),
        scratch_shapes=[
            pltpu.VMEM((N_DEV, ROWS, D_OUT), jnp.bfloat16),
            pltpu.VMEM((N_DEV, ROWS, D_OUT), jnp.bfloat16),
            pltpu.VMEM((N_TOK, E_LOCAL), jnp.float32),
            pltpu.SemaphoreType.DMA((N_DEV,)),
            pltpu.SemaphoreType.DMA((N_DEV,)),
        ],
        compiler_params=pltpu.CompilerParams(collective_id=0),
    )(x, router_W, route_idx, expert_W, shared_W)
